# baseline (speedup 1.0000x reference)
import functools

import jax
import jax.numpy as jnp
from jax import lax
from jax.experimental import pallas as pl
from jax.experimental.pallas import tpu as pltpu
from jax.experimental.pallas import tpu_sc as plsc

N = 10000
E = 160000
B = 16
HD = 128

NC = 2
NS = 16
NW = NC * NS

CH = 96
CPW0 = 94
CPW1 = 14
E_PAD = NS * (CPW0 + CPW1) * CH
NCHUNKS = E_PAD // CH
N_ACC = 10112
ZPW = N_ACC // NS

NB = 1000
EB = 2048

_DOT = functools.partial(jnp.dot, precision=lax.Precision.HIGHEST)


def _dotT(a, b):
    return lax.dot_general(a, b, (((0,), (0,)), ((), ())),
                           precision=lax.Precision.HIGHEST)


def _lrelu(x):
    return jnp.maximum(x, 0.1 * x)


def _edge_sc_body(v_hbm, ke_hbm, src_hbm, dst_hbm, out_hbm,
                  isrc, idst, rows0, kev0, rows1, kev1, sve_sh,
                  isem0, isem1, lsem0, lsem1, ssem0, ssem1):
    c = lax.axis_index("c")
    s = lax.axis_index("s")
    wid = c * NS + s

    def zrow(r, _):
        for j in range(8):
            rows0[r, pl.ds(j * 16, 16)] = jnp.zeros((16,), jnp.float32)
        return 0
    lax.fori_loop(0, CH, zrow, 0)
    zrem = ZPW - (ZPW // CH) * CH
    for j in range(ZPW // CH):
        pltpu.sync_copy(rows0, sve_sh.at[pl.ds(s * ZPW + j * CH, CH)])
    if zrem:
        pltpu.sync_copy(rows0.at[pl.ds(0, zrem)],
                        sve_sh.at[pl.ds(s * ZPW + (ZPW // CH) * CH, zrem)])
    plsc.subcore_barrier()

    cpw = lax.select(c == 0, jnp.int32(CPW0), jnp.int32(CPW1))
    cbase = c * NS * CPW0 + s * cpw
    isems = (isem0, isem1)

    def issue_src(g, b):
        pltpu.async_copy(src_hbm.at[cbase + g], isrc.at[b], isems[b])

    def wait_src(g, b):
        pltpu.make_async_copy(src_hbm.at[cbase + g], isrc.at[b],
                              isems[b]).wait()

    def issue_dst(g, b):
        pltpu.async_copy(dst_hbm.at[cbase + g], idst.at[b], isems[b])

    def wait_dst(g, b):
        pltpu.make_async_copy(dst_hbm.at[cbase + g], idst.at[b],
                              isems[b]).wait()

    def issue_data(g, b, rows, kev, lsem):
        pltpu.async_copy(v_hbm.at[isrc.at[b]], rows, lsem)
        pltpu.async_copy(ke_hbm.at[pl.ds((cbase + g) * CH, CH)], kev, lsem)

    def wait_data(g, b, rows, kev, lsem):
        pltpu.make_async_copy(v_hbm.at[isrc.at[b]], rows, lsem).wait()
        pltpu.make_async_copy(
            ke_hbm.at[pl.ds((cbase + g) * CH, CH)], kev, lsem).wait()

    npair = cpw // 2

    def issue_scatter(b, rows, ssem):
        pltpu.async_copy(rows, sve_sh.at[idst.at[b]], ssem, add=True)

    def wait_scatter(b, rows, ssem):
        pltpu.make_async_copy(rows, sve_sh.at[idst.at[b]], ssem).wait()

    def compute(rows, kev):
        @plsc.parallel_loop(0, CH, unroll=2)
        def _(r):
            for j in range(4):
                ki = kev[r, pl.ds(j * 16, 16)]
                ka = lax.bitcast_convert_type(ki << 16, jnp.float32)
                kb = lax.bitcast_convert_type(ki & jnp.int32(-65536),
                                              jnp.float32)
                t = rows[r, pl.ds(j * 32, 16)] * ka
                rows[r, pl.ds(j * 32, 16)] = jnp.maximum(t, t * 0.1)
                t = rows[r, pl.ds(j * 32 + 16, 16)] * kb
                rows[r, pl.ds(j * 32 + 16, 16)] = jnp.maximum(t, t * 0.1)

    issue_src(0, 0)
    wait_src(0, 0)
    issue_dst(0, 0)
    issue_data(0, 0, rows0, kev0, lsem0)
    issue_src(1, 1)

    def pair(i, _):
        g0 = 2 * i
        g1 = g0 + 1
        last = npair - 1

        @pl.when(i > 0)
        def _():
            wait_scatter(1, rows1, ssem1)
        wait_src(g1, 1)
        issue_data(g1, 1, rows1, kev1, lsem1)
        issue_dst(g1, 1)
        wait_data(g0, 0, rows0, kev0, lsem0)
        wait_dst(g0, 0)

        @pl.when(i < last)
        def _():
            issue_src(g0 + 2, 0)
        compute(rows0, kev0)
        issue_scatter(0, rows0, ssem0)

        wait_scatter(0, rows0, ssem0)

        @pl.when(i < last)
        def _():
            wait_src(g0 + 2, 0)
            issue_data(g0 + 2, 0, rows0, kev0, lsem0)
            issue_dst(g0 + 2, 0)
        wait_data(g1, 1, rows1, kev1, lsem1)
        wait_dst(g1, 1)

        @pl.when(i < last)
        def _():
            issue_src(g1 + 2, 1)
        compute(rows1, kev1)
        issue_scatter(1, rows1, ssem1)
        return 0

    lax.fori_loop(0, npair, pair, 0)
    wait_scatter(1, rows1, ssem1)
    plsc.subcore_barrier()

    pltpu.sync_copy(sve_sh.at[pl.ds(s * ZPW, ZPW)],
                    out_hbm.at[c, pl.ds(s * ZPW, ZPW)])


def _edge_aggregate(v, ke_pad, src_3d, dst_3d):
    ke_pad = lax.bitcast_convert_type(
        ke_pad.reshape(E_PAD, 64, 2), jnp.int32)
    mesh = plsc.VectorSubcoreMesh(core_axis_name="c", subcore_axis_name="s")
    return pl.kernel(
        _edge_sc_body,
        out_type=jax.ShapeDtypeStruct((NC, N_ACC, 128), jnp.float32),
        mesh=mesh,
        scratch_types=[
            pltpu.VMEM((2, CH), jnp.int32),
            pltpu.VMEM((2, CH), jnp.int32),
            pltpu.VMEM((CH, 128), jnp.float32),
            pltpu.VMEM((CH, 64), jnp.int32),
            pltpu.VMEM((CH, 128), jnp.float32),
            pltpu.VMEM((CH, 64), jnp.int32),
            pltpu.VMEM_SHARED((N_ACC, 128), jnp.float32),
            pltpu.SemaphoreType.DMA,
            pltpu.SemaphoreType.DMA,
            pltpu.SemaphoreType.DMA,
            pltpu.SemaphoreType.DMA,
            pltpu.SemaphoreType.DMA,
            pltpu.SemaphoreType.DMA,
        ],
    )(v, ke_pad, src_3d, dst_3d)


def _init_body(x_ref, oh_ref, w0_ref, w1_ref, w2_ref, bias_ref, g_ref, b_ref,
               v_ref, ssum_ref, cnt_ref):
    x = x_ref[...]
    oh = oh_ref[...]
    v = _DOT(x, w0_ref[...]) + bias_ref[0:1, :]
    h = jnp.maximum(_DOT(v, w1_ref[...]) + bias_ref[1:2, :64], 0.0)
    v = _DOT(h, w2_ref[...]) + bias_ref[2:3, :]
    m = jnp.mean(v, -1, keepdims=True)
    var = jnp.mean((v - m) ** 2, -1, keepdims=True)
    v = (v - m) / jnp.sqrt(var + 1e-5) * g_ref[0:1, :] + b_ref[0:1, :]
    v_ref[...] = v

    @pl.when(pl.program_id(0) == 0)
    def _():
        ssum_ref[...] = jnp.zeros_like(ssum_ref)
        cnt_ref[...] = jnp.zeros_like(cnt_ref)
    ssum_ref[...] += _dotT(oh, v)
    cnt_ref[...] += _dotT(oh, jnp.ones_like(v))


def _node_init(x_pad, oh, w0, w1, w2, bias, g, b):
    grid = N // NB
    return pl.pallas_call(
        _init_body,
        grid=(grid,),
        in_specs=[
            pl.BlockSpec((NB, 384), lambda i: (i, 0)),
            pl.BlockSpec((NB, 16), lambda i: (i, 0)),
            pl.BlockSpec((384, 128), lambda i: (0, 0)),
            pl.BlockSpec((128, 64), lambda i: (0, 0)),
            pl.BlockSpec((64, 128), lambda i: (0, 0)),
            pl.BlockSpec((8, 128), lambda i: (0, 0)),
            pl.BlockSpec((1, 128), lambda i: (0, 0)),
            pl.BlockSpec((1, 128), lambda i: (0, 0)),
        ],
        out_specs=[
            pl.BlockSpec((NB, 128), lambda i: (i, 0)),
            pl.BlockSpec((16, 128), lambda i: (0, 0)),
            pl.BlockSpec((16, 128), lambda i: (0, 0)),
        ],
        out_shape=[
            jax.ShapeDtypeStruct((N, 128), jnp.float32),
            jax.ShapeDtypeStruct((16, 128), jnp.float32),
            jax.ShapeDtypeStruct((16, 128), jnp.float32),
        ],
    )(x_pad, oh, w0, w1, w2, bias, g, b)


def _ke_body(ef_ref, m_ref, c_ref, ke1_ref, ke2_ref, ke3_ref):
    ef = ef_ref[...]
    rid = pl.program_id(0) * EB + lax.broadcasted_iota(jnp.int32, (EB, 1), 0)
    valid = rid < E
    for l, out in enumerate((ke1_ref, ke2_ref, ke3_ref)):
        ke = _DOT(ef, m_ref[l]) + c_ref[l:l + 1, :]
        out[...] = jnp.where(valid, ke, 0.0).astype(jnp.bfloat16)


def _ke_all(ef_pad, m_stack, c_stack):
    grid = E_PAD // EB
    shp = jax.ShapeDtypeStruct((E_PAD, 128), jnp.bfloat16)
    return pl.pallas_call(
        _ke_body,
        grid=(grid,),
        in_specs=[
            pl.BlockSpec((EB, 16), lambda i: (i, 0)),
            pl.BlockSpec((3, 16, 128), lambda i: (0, 0, 0)),
            pl.BlockSpec((3, 128), lambda i: (0, 0)),
        ],
        out_specs=[pl.BlockSpec((EB, 128), lambda i: (i, 0))] * 3,
        out_shape=[shp, shp, shp],
    )(ef_pad, m_stack, c_stack)


def _sprep_body(ssum_ref, cnt_ref, wa_ref, wb_ref, wc_ref, bias_ref,
                s_ref, s2s_ref, dsup_ref, s2mg_ref, k_head):
    s = ssum_ref[...] / jnp.maximum(cnt_ref[...], 1.0)
    s_ref[...] = s
    s2s_ref[...] = jnp.tanh(_DOT(s, wa_ref[...]) + bias_ref[0:1, :])
    s2mg_ref[...] = jnp.tanh(_DOT(s, wc_ref[...]) + bias_ref[1:2, :])
    for h in range(k_head):
        dsup_ref[h * 16:(h + 1) * 16, :] = jnp.tanh(
            _DOT(s, wb_ref[h]) + bias_ref[2 + h:3 + h, :])


def _sprep(ssum, cnt, wa, wb_stack, wc, bias, k_head):
    kk = k_head * 16
    return pl.pallas_call(
        functools.partial(_sprep_body, k_head=k_head),
        grid=(1,),
        in_specs=[
            pl.BlockSpec((16, 128), lambda i: (0, 0)),
            pl.BlockSpec((16, 128), lambda i: (0, 0)),
            pl.BlockSpec((128, 128), lambda i: (0, 0)),
            pl.BlockSpec((k_head, 128, 128), lambda i: (0, 0, 0)),
            pl.BlockSpec((128, 128), lambda i: (0, 0)),
            pl.BlockSpec((8, 128), lambda i: (0, 0)),
        ],
        out_specs=[
            pl.BlockSpec((16, 128), lambda i: (0, 0)),
            pl.BlockSpec((16, 128), lambda i: (0, 0)),
            pl.BlockSpec((kk, 128), lambda i: (0, 0)),
            pl.BlockSpec((16, 128), lambda i: (0, 0)),
        ],
        out_shape=[
            jax.ShapeDtypeStruct((16, 128), jnp.float32),
            jax.ShapeDtypeStruct((16, 128), jnp.float32),
            jax.ShapeDtypeStruct((kk, 128), jnp.float32),
            jax.ShapeDtypeStruct((16, 128), jnp.float32),
        ],
    )(ssum, cnt, wa, wb_stack, wc, bias)


def _p1_body(v_ref, oh_ref, wa_ref, cvec_ref, dsup_ref, bias_ref,
             amax_ref, k_head):
    v = v_ref[...]
    oh = oh_ref[...]
    @pl.when(pl.program_id(0) == 0)
    def _():
        amax_ref[...] = jnp.full_like(amax_ref, -1e30)
    cols = []
    for h in range(k_head):
        dn = jnp.tanh(_DOT(v, wa_ref[h]) + bias_ref[h:h + 1, :])
        dsn = _DOT(oh, dsup_ref[h * 16:(h + 1) * 16, :])
        a = jnp.sum(dn * dsn * cvec_ref[h:h + 1, :], axis=1, keepdims=True) \
            + bias_ref[4 + h:5 + h, 0:1]
        masked = jnp.where(oh > 0.0, a, -1e30)
        cols.append(jnp.max(masked, axis=0, keepdims=True))
    block = jnp.concatenate(cols + [jnp.full((8 - k_head, 16), -1e30)], axis=0)
    block = jnp.pad(block, ((0, 0), (0, 112)), constant_values=-1e30)
    amax_ref[...] = jnp.maximum(amax_ref[...], block)


def _p2_body(v_ref, oh_ref, wa_ref, cvec_ref, dsup_ref, wd_ref, amax_ref,
             bias_ref, num_ref, den_ref, k_head):
    v = v_ref[...]
    oh = oh_ref[...]
    @pl.when(pl.program_id(0) == 0)
    def _():
        num_ref[...] = jnp.zeros_like(num_ref)
        den_ref[...] = jnp.zeros_like(den_ref)
    den_cols = []
    for h in range(k_head):
        dn = jnp.tanh(_DOT(v, wa_ref[h]) + bias_ref[h:h + 1, :])
        dsn = _DOT(oh, dsup_ref[h * 16:(h + 1) * 16, :])
        a = jnp.sum(dn * dsn * cvec_ref[h:h + 1, :], axis=1, keepdims=True) \
            + bias_ref[4 + h:5 + h, 0:1]
        amax_n = _DOT(oh, amax_ref[h, 0:16])[:, None]
        ex = jnp.exp(a - amax_n)
        lind = _DOT(v, wd_ref[h]) + bias_ref[8 + h:9 + h, :]
        num_ref[h * 16:(h + 1) * 16, :] += _dotT(oh, ex * lind)
        den_cols.append(_dotT(oh, ex))
    den = jnp.concatenate(
        den_cols + [jnp.zeros((16, 128 - k_head), jnp.float32)], axis=1)
    den_ref[...] += den


def _helper_pass(v, oh, wa_stack, cvec, dsup, wd_stack, bias, k_head):
    grid = N // NB
    kk = k_head * 16
    amax = pl.pallas_call(
        functools.partial(_p1_body, k_head=k_head),
        grid=(grid,),
        in_specs=[
            pl.BlockSpec((NB, 128), lambda i: (i, 0)),
            pl.BlockSpec((NB, 16), lambda i: (i, 0)),
            pl.BlockSpec((k_head, 128, 128), lambda i: (0, 0, 0)),
            pl.BlockSpec((8, 128), lambda i: (0, 0)),
            pl.BlockSpec((kk, 128), lambda i: (0, 0)),
            pl.BlockSpec((16, 128), lambda i: (0, 0)),
        ],
        out_specs=pl.BlockSpec((8, 128), lambda i: (0, 0)),
        out_shape=jax.ShapeDtypeStruct((8, 128), jnp.float32),
    )(v, oh, wa_stack, cvec, dsup, bias)
    num, den = pl.pallas_call(
        functools.partial(_p2_body, k_head=k_head),
        grid=(grid,),
        in_specs=[
            pl.BlockSpec((NB, 128), lambda i: (i, 0)),
            pl.BlockSpec((NB, 16), lambda i: (i, 0)),
            pl.BlockSpec((k_head, 128, 128), lambda i: (0, 0, 0)),
            pl.BlockSpec((8, 128), lambda i: (0, 0)),
            pl.BlockSpec((kk, 128), lambda i: (0, 0)),
            pl.BlockSpec((k_head, 128, 128), lambda i: (0, 0, 0)),
            pl.BlockSpec((8, 128), lambda i: (0, 0)),
            pl.BlockSpec((16, 128), lambda i: (0, 0)),
        ],
        out_specs=[
            pl.BlockSpec((kk, 128), lambda i: (0, 0)),
            pl.BlockSpec((16, 128), lambda i: (0, 0)),
        ],
        out_shape=[
            jax.ShapeDtypeStruct((kk, 128), jnp.float32),
            jax.ShapeDtypeStruct((16, 128), jnp.float32),
        ],
    )(v, oh, wa_stack, cvec, dsup, wd_stack, amax, bias)
    return num, den


def _gates_body(v_ref, sve0_ref, sve1_ref, oh_ref, s2mg_ref,
                wes_ref, wev_ref, wa_ref, wb_ref, wih_ref, whh_ref,
                bias_ref, vv_ref):
    v = v_ref[...]
    oh = oh_ref[...]
    sve = sve0_ref[0] + sve1_ref[0]
    m2m = _lrelu(_DOT(sve, wes_ref[...]) + _DOT(v, wev_ref[...])
                 + bias_ref[0:1, :])
    s2m = _DOT(oh, s2mg_ref[...])
    z = jax.nn.sigmoid(_DOT(m2m, wa_ref[...]) + _DOT(s2m, wb_ref[...])
                       + bias_ref[1:2, :])
    h = z * s2m + (1.0 - z) * m2m
    bih = jnp.concatenate([bias_ref[2:3, :], bias_ref[3:4, :],
                           bias_ref[4:5, :]], axis=1)
    bhh = jnp.concatenate([bias_ref[5:6, :], bias_ref[6:7, :],
                           bias_ref[7:8, :]], axis=1)
    gi = _DOT(v, wih_ref[...]) + bih
    gh = _DOT(h, whh_ref[...]) + bhh
    r = jax.nn.sigmoid(gi[:, 0:128] + gh[:, 0:128])
    z2 = jax.nn.sigmoid(gi[:, 128:256] + gh[:, 128:256])
    n = jnp.tanh(gi[:, 256:384] + r * gh[:, 256:384])
    vv_ref[...] = (1.0 - z2) * n + z2 * h


def _gates(v, parts, oh, s2mg, wes, wev, wa, wb, wih, whh, bias):
    grid = N // NB
    return pl.pallas_call(
        _gates_body,
        grid=(grid,),
        in_specs=[
            pl.BlockSpec((NB, 128), lambda i: (i, 0)),
            pl.BlockSpec((1, NB, 128), lambda i: (0, i, 0)),
            pl.BlockSpec((1, NB, 128), lambda i: (1, i, 0)),
            pl.BlockSpec((NB, 16), lambda i: (i, 0)),
            pl.BlockSpec((16, 128), lambda i: (0, 0)),
            pl.BlockSpec((128, 128), lambda i: (0, 0)),
            pl.BlockSpec((128, 128), lambda i: (0, 0)),
            pl.BlockSpec((128, 128), lambda i: (0, 0)),
            pl.BlockSpec((128, 128), lambda i: (0, 0)),
            pl.BlockSpec((128, 384), lambda i: (0, 0)),
            pl.BlockSpec((128, 384), lambda i: (0, 0)),
            pl.BlockSpec((8, 128), lambda i: (0, 0)),
        ],
        out_specs=pl.BlockSpec((NB, 128), lambda i: (i, 0)),
        out_shape=jax.ShapeDtypeStruct((N, 128), jnp.float32),
    )(v, parts, parts, oh, s2mg, wes, wev, wa, wb, wih, whh, bias)


def _supdate_body(s_ref, s2s_ref, num_ref, den_ref, wm_ref, wa_ref, wb_ref,
                  wih_ref, whh_ref, bias_ref, ss_ref, k_head):
    s = s_ref[...]
    s2s = s2s_ref[...]
    houts = []
    for h in range(k_head):
        d = jnp.maximum(den_ref[:, h:h + 1], 1e-30)
        houts.append(num_ref[h * 16:(h + 1) * 16, :] / d)
    mcat = jnp.concatenate(houts, axis=1)
    m2s = jnp.tanh(_DOT(mcat, wm_ref[...]) + bias_ref[0:1, :])
    z = jax.nn.sigmoid(_DOT(s2s, wa_ref[...]) + _DOT(m2s, wb_ref[...])
                       + bias_ref[1:2, :])
    h = z * m2s + (1.0 - z) * s2s
    bih = jnp.concatenate([bias_ref[2:3, :], bias_ref[3:4, :],
                           bias_ref[4:5, :]], axis=1)
    bhh = jnp.concatenate([bias_ref[5:6, :], bias_ref[6:7, :],
                           bias_ref[7:8, :]], axis=1)
    gi = _DOT(s, wih_ref[...]) + bih
    gh = _DOT(h, whh_ref[...]) + bhh
    r = jax.nn.sigmoid(gi[:, 0:128] + gh[:, 0:128])
    z2 = jax.nn.sigmoid(gi[:, 128:256] + gh[:, 128:256])
    n = jnp.tanh(gi[:, 256:384] + r * gh[:, 256:384])
    ss_ref[...] = (1.0 - z2) * n + z2 * h


def _supdate(s, s2s, num, den, wm, wa, wb, wih, whh, bias, k_head):
    kk = k_head * 16
    return pl.pallas_call(
        functools.partial(_supdate_body, k_head=k_head),
        grid=(1,),
        in_specs=[
            pl.BlockSpec((16, 128), lambda i: (0, 0)),
            pl.BlockSpec((16, 128), lambda i: (0, 0)),
            pl.BlockSpec((kk, 128), lambda i: (0, 0)),
            pl.BlockSpec((16, 128), lambda i: (0, 0)),
            pl.BlockSpec((k_head * 128, 128), lambda i: (0, 0)),
            pl.BlockSpec((128, 128), lambda i: (0, 0)),
            pl.BlockSpec((128, 128), lambda i: (0, 0)),
            pl.BlockSpec((128, 384), lambda i: (0, 0)),
            pl.BlockSpec((128, 384), lambda i: (0, 0)),
            pl.BlockSpec((8, 128), lambda i: (0, 0)),
        ],
        out_specs=pl.BlockSpec((16, 128), lambda i: (0, 0)),
        out_shape=jax.ShapeDtypeStruct((16, 128), jnp.float32),
    )(s, s2s, num, den, wm, wa, wb, wih, whh, bias)


def _prep_weights(params):
    P = {}
    p0 = params["a_init"]
    w0 = jnp.pad(p0["W"].T, ((0, 2), (0, 0)))
    w1 = params["a_init1"]["W"].T
    w2 = params["a_init2"]["W"].T
    bias = jnp.zeros((8, 128), jnp.float32)
    bias = bias.at[0].set(p0["b"])
    bias = bias.at[1, :64].set(params["a_init1"]["b"])
    bias = bias.at[2].set(params["a_init2"]["b"])
    P["init"] = (w0, w1, w2, bias,
                 params["norml"]["g"][None, :], params["norml"]["b"][None, :])

    wb = params["b_init"]
    perm = jnp.arange(128).reshape(4, 2, 16).transpose(0, 2, 1).reshape(-1)
    m_stack, c_stack = [], []
    for name in ("conv1", "conv2", "conv3"):
        K = params[name]["K"]
        m = (K["W"] @ wb["W"]).T
        m_stack.append(jnp.pad(m[:, perm], ((0, 4), (0, 0))))
        c_stack.append((wb["b"] @ K["W"].T + K["b"])[perm])
    P["ke"] = (jnp.stack(m_stack), jnp.stack(c_stack))

    for name in ("conv1", "conv2", "conv3"):
        p = params[name]
        k_head = len(p["helpers"])
        L = {}
        sb = jnp.zeros((8, 128), jnp.float32)
        sb = sb.at[0].set(p["A"]["b"])
        sb = sb.at[1].set(p["C"]["b"])
        for h, hp in enumerate(p["helpers"]):
            sb = sb.at[2 + h].set(hp["B"]["b"])
        L["sprep"] = (p["A"]["W"].T,
                      jnp.stack([hp["B"]["W"].T for hp in p["helpers"]]),
                      p["C"]["W"].T, sb, k_head)

        hb = jnp.zeros((16, 128), jnp.float32)
        cvec = jnp.zeros((8, 128), jnp.float32)
        for h, hp in enumerate(p["helpers"]):
            hb = hb.at[h].set(hp["A"]["b"])
            hb = hb.at[4 + h].set(jnp.broadcast_to(hp["C"]["b"], (128,)))
            hb = hb.at[8 + h].set(hp["D"]["b"])
            cvec = cvec.at[h].set(hp["C"]["W"][0])
        L["helper"] = (jnp.stack([hp["A"]["W"].T for hp in p["helpers"]]),
                       cvec,
                       jnp.stack([hp["D"]["W"].T for hp in p["helpers"]]),
                       hb, k_head)

        gm = p["gm"]
        gb = jnp.zeros((8, 128), jnp.float32)
        gb = gb.at[0].set(p["E"]["b"])
        gb = gb.at[1].set(gm["A"]["b"] + gm["B"]["b"])
        for t in range(3):
            gb = gb.at[2 + t].set(gm["bih"][t * 128:(t + 1) * 128])
            gb = gb.at[5 + t].set(gm["bhh"][t * 128:(t + 1) * 128])
        we = p["E"]["W"].T
        L["gates"] = (we[0:128], we[128:256], gm["A"]["W"].T, gm["B"]["W"].T,
                      gm["Wih"].T, gm["Whh"].T, gb)

        gs = p["gs"]
        ub = jnp.zeros((8, 128), jnp.float32)
        ub = ub.at[0].set(p["B"]["b"])
        ub = ub.at[1].set(gs["A"]["b"] + gs["B"]["b"])
        for t in range(3):
            ub = ub.at[2 + t].set(gs["bih"][t * 128:(t + 1) * 128])
            ub = ub.at[5 + t].set(gs["bhh"][t * 128:(t + 1) * 128])
        L["supdate"] = (p["B"]["W"].T, gs["A"]["W"].T, gs["B"]["W"].T,
                        gs["Wih"].T, gs["Whh"].T, ub, k_head)
        P[name] = L
    return P


def kernel(x, edge_feat, edge_index, graph_ids, params):
    src_pad = jnp.pad(edge_index[0], (0, E_PAD - E)).reshape(NCHUNKS, CH)
    dst_pad = jnp.pad(edge_index[1], (0, E_PAD - E)).reshape(NCHUNKS, CH)
    ef_pad = jnp.pad(edge_feat, ((0, E_PAD - E), (0, 4)))
    x_pad = jnp.pad(x, ((0, 0), (0, 2)))
    oh = (graph_ids[:, None] == jnp.arange(16)[None, :]).astype(jnp.float32)

    P = _prep_weights(params)
    v, ssum, cnt = _node_init(x_pad, oh, *P["init"])
    kes = _ke_all(ef_pad, *P["ke"])

    sraw, craw = ssum, cnt
    for li, name in enumerate(("conv1", "conv2", "conv3")):
        L = P[name]
        s, s2s, dsup, s2mg = _sprep(sraw, craw, *L["sprep"])
        wa_stack, cvec, wd_stack, hbias, k_head = L["helper"]
        num, den = _helper_pass(v, oh, wa_stack, cvec, dsup, wd_stack,
                                hbias, k_head)
        parts = _edge_aggregate(v, kes[li], src_pad, dst_pad)
        vv = _gates(v, parts, oh, s2mg, *L["gates"])
        ss = _supdate(s, s2s, num, den, *L["supdate"])
        v, sraw, craw = vv, ss, jnp.ones((16, 128), jnp.float32)
    return v


# speedup vs baseline: 1.4804x; 1.4804x over previous
import functools

import jax
import jax.numpy as jnp
from jax import lax
from jax.experimental import pallas as pl
from jax.experimental.pallas import tpu as pltpu
from jax.experimental.pallas import tpu_sc as plsc

N = 10000
E = 160000
B = 16
HD = 128

NC = 2
NS = 16
NW = NC * NS

CH = 96
CPW0 = 94
CPW1 = 14
E_PAD = NS * (CPW0 + CPW1) * CH
NCHUNKS = E_PAD // CH
N_ACC = 10112
ZPW = N_ACC // NS

NB = 1000
EB = 2048

_DOT = functools.partial(jnp.dot, precision=lax.Precision.HIGHEST)


def _dotT(a, b):
    return lax.dot_general(a, b, (((0,), (0,)), ((), ())),
                           precision=lax.Precision.HIGHEST)


def _lrelu(x):
    return jnp.maximum(x, 0.1 * x)


def _edge_sc_body(v_hbm, ke_hbm, src_hbm, dst_hbm, out_hbm,
                  isrc, idst, rows0, kev0, rows1, kev1, sve_sh,
                  isem0, isem1, lsem0, lsem1, ssem0, ssem1):
    c = lax.axis_index("c")
    s = lax.axis_index("s")
    wid = c * NS + s

    def zrow(r, _):
        for j in range(8):
            rows0[r, pl.ds(j * 16, 16)] = jnp.zeros((16,), jnp.float32)
        return 0
    lax.fori_loop(0, CH, zrow, 0)
    zrem = ZPW - (ZPW // CH) * CH
    for j in range(ZPW // CH):
        pltpu.sync_copy(rows0, sve_sh.at[pl.ds(s * ZPW + j * CH, CH)])
    if zrem:
        pltpu.sync_copy(rows0.at[pl.ds(0, zrem)],
                        sve_sh.at[pl.ds(s * ZPW + (ZPW // CH) * CH, zrem)])
    plsc.subcore_barrier()

    cpw = lax.select(c == 0, jnp.int32(CPW0), jnp.int32(CPW1))
    cbase = c * NS * CPW0 + s * cpw
    isems = (isem0, isem1)

    def issue_src(g, b):
        pltpu.async_copy(src_hbm.at[cbase + g], isrc.at[b], isems[b])

    def wait_src(g, b):
        pltpu.make_async_copy(src_hbm.at[cbase + g], isrc.at[b],
                              isems[b]).wait()

    def issue_dst(g, b):
        pltpu.async_copy(dst_hbm.at[cbase + g], idst.at[b], isems[b])

    def wait_dst(g, b):
        pltpu.make_async_copy(dst_hbm.at[cbase + g], idst.at[b],
                              isems[b]).wait()

    def issue_data(g, b, rows, kev, lsem):
        pltpu.async_copy(v_hbm.at[isrc.at[b]], rows, lsem)
        pltpu.async_copy(ke_hbm.at[pl.ds((cbase + g) * CH, CH)], kev, lsem)

    def wait_data(g, b, rows, kev, lsem):
        pltpu.make_async_copy(v_hbm.at[isrc.at[b]], rows, lsem).wait()
        pltpu.make_async_copy(
            ke_hbm.at[pl.ds((cbase + g) * CH, CH)], kev, lsem).wait()

    npair = cpw // 2

    def issue_scatter(b, rows, ssem):
        pltpu.async_copy(rows, sve_sh.at[idst.at[b]], ssem, add=True)

    def wait_scatter(b, rows, ssem):
        pltpu.make_async_copy(rows, sve_sh.at[idst.at[b]], ssem).wait()

    def compute(rows, kev):
        @plsc.parallel_loop(0, CH, unroll=2)
        def _(r):
            for j in range(4):
                ki = kev[r, pl.ds(j * 16, 16)]
                ka = lax.bitcast_convert_type(ki << 16, jnp.float32)
                kb = lax.bitcast_convert_type(ki & jnp.int32(-65536),
                                              jnp.float32)
                t = rows[r, pl.ds(j * 32, 16)] * ka
                rows[r, pl.ds(j * 32, 16)] = jnp.maximum(t, t * 0.1)
                t = rows[r, pl.ds(j * 32 + 16, 16)] * kb
                rows[r, pl.ds(j * 32 + 16, 16)] = jnp.maximum(t, t * 0.1)

    issue_src(0, 0)
    wait_src(0, 0)
    issue_dst(0, 0)
    issue_data(0, 0, rows0, kev0, lsem0)
    issue_src(1, 1)

    def pair(i, _):
        g0 = 2 * i
        g1 = g0 + 1
        last = npair - 1

        @pl.when(i > 0)
        def _():
            wait_scatter(1, rows1, ssem1)
        wait_src(g1, 1)
        issue_data(g1, 1, rows1, kev1, lsem1)
        issue_dst(g1, 1)
        wait_data(g0, 0, rows0, kev0, lsem0)
        wait_dst(g0, 0)

        @pl.when(i < last)
        def _():
            issue_src(g0 + 2, 0)
        compute(rows0, kev0)
        issue_scatter(0, rows0, ssem0)

        wait_scatter(0, rows0, ssem0)

        @pl.when(i < last)
        def _():
            wait_src(g0 + 2, 0)
            issue_data(g0 + 2, 0, rows0, kev0, lsem0)
            issue_dst(g0 + 2, 0)
        wait_data(g1, 1, rows1, kev1, lsem1)
        wait_dst(g1, 1)

        @pl.when(i < last)
        def _():
            issue_src(g1 + 2, 1)
        compute(rows1, kev1)
        issue_scatter(1, rows1, ssem1)
        return 0

    lax.fori_loop(0, npair, pair, 0)
    wait_scatter(1, rows1, ssem1)
    plsc.subcore_barrier()

    pltpu.sync_copy(sve_sh.at[pl.ds(s * ZPW, ZPW)],
                    out_hbm.at[c, pl.ds(s * ZPW, ZPW)])


def _edge_aggregate(v, ke_pad, src_3d, dst_3d):
    mesh = plsc.VectorSubcoreMesh(core_axis_name="c", subcore_axis_name="s")
    return pl.kernel(
        _edge_sc_body,
        out_type=jax.ShapeDtypeStruct((NC, N_ACC, 128), jnp.float32),
        mesh=mesh,
        scratch_types=[
            pltpu.VMEM((2, CH), jnp.int32),
            pltpu.VMEM((2, CH), jnp.int32),
            pltpu.VMEM((CH, 128), jnp.float32),
            pltpu.VMEM((CH, 64), jnp.int32),
            pltpu.VMEM((CH, 128), jnp.float32),
            pltpu.VMEM((CH, 64), jnp.int32),
            pltpu.VMEM_SHARED((N_ACC, 128), jnp.float32),
            pltpu.SemaphoreType.DMA,
            pltpu.SemaphoreType.DMA,
            pltpu.SemaphoreType.DMA,
            pltpu.SemaphoreType.DMA,
            pltpu.SemaphoreType.DMA,
            pltpu.SemaphoreType.DMA,
        ],
    )(v, ke_pad, src_3d, dst_3d)


def _init_body(x_ref, oh_ref, w0_ref, w1_ref, w2_ref, bias_ref, g_ref, b_ref,
               v_ref, ssum_ref, cnt_ref):
    x = x_ref[...]
    oh = oh_ref[...]
    v = _DOT(x, w0_ref[...]) + bias_ref[0:1, :]
    h = jnp.maximum(_DOT(v, w1_ref[...]) + bias_ref[1:2, :64], 0.0)
    v = _DOT(h, w2_ref[...]) + bias_ref[2:3, :]
    m = jnp.mean(v, -1, keepdims=True)
    var = jnp.mean((v - m) ** 2, -1, keepdims=True)
    v = (v - m) / jnp.sqrt(var + 1e-5) * g_ref[0:1, :] + b_ref[0:1, :]
    v_ref[...] = v

    @pl.when(pl.program_id(0) == 0)
    def _():
        ssum_ref[...] = jnp.zeros_like(ssum_ref)
        cnt_ref[...] = jnp.zeros_like(cnt_ref)
    ssum_ref[...] += _dotT(oh, v)
    cnt_ref[...] += _dotT(oh, jnp.ones_like(v))


def _node_init(x_pad, oh, w0, w1, w2, bias, g, b):
    grid = N // NB
    return pl.pallas_call(
        _init_body,
        grid=(grid,),
        in_specs=[
            pl.BlockSpec((NB, 384), lambda i: (i, 0)),
            pl.BlockSpec((NB, 16), lambda i: (i, 0)),
            pl.BlockSpec((384, 128), lambda i: (0, 0)),
            pl.BlockSpec((128, 64), lambda i: (0, 0)),
            pl.BlockSpec((64, 128), lambda i: (0, 0)),
            pl.BlockSpec((8, 128), lambda i: (0, 0)),
            pl.BlockSpec((1, 128), lambda i: (0, 0)),
            pl.BlockSpec((1, 128), lambda i: (0, 0)),
        ],
        out_specs=[
            pl.BlockSpec((NB, 128), lambda i: (i, 0)),
            pl.BlockSpec((16, 128), lambda i: (0, 0)),
            pl.BlockSpec((16, 128), lambda i: (0, 0)),
        ],
        out_shape=[
            jax.ShapeDtypeStruct((N, 128), jnp.float32),
            jax.ShapeDtypeStruct((16, 128), jnp.float32),
            jax.ShapeDtypeStruct((16, 128), jnp.float32),
        ],
    )(x_pad, oh, w0, w1, w2, bias, g, b)


def _bf16_bits(x):
    b = lax.bitcast_convert_type(x.astype(jnp.bfloat16), jnp.uint16)
    return b.astype(jnp.int32)


def _ke_body(ef_ref, m_ref, c_ref, ke1_ref, ke2_ref, ke3_ref):
    ef = ef_ref[...]
    rid = pl.program_id(0) * EB + lax.broadcasted_iota(jnp.int32, (EB, 1), 0)
    valid = rid < E
    for l, out in enumerate((ke1_ref, ke2_ref, ke3_ref)):
        lo = _DOT(ef, m_ref[l, :, 0:64]) + c_ref[l:l + 1, 0:64]
        hi = _DOT(ef, m_ref[l, :, 64:128]) + c_ref[l:l + 1, 64:128]
        packed = _bf16_bits(lo) | (_bf16_bits(hi) << 16)
        out[...] = jnp.where(valid, packed, 0)


def _ke_all(ef_pad, m_stack, c_stack):
    grid = E_PAD // EB
    shp = jax.ShapeDtypeStruct((E_PAD, 64), jnp.int32)
    return pl.pallas_call(
        _ke_body,
        grid=(grid,),
        in_specs=[
            pl.BlockSpec((EB, 16), lambda i: (i, 0)),
            pl.BlockSpec((3, 16, 128), lambda i: (0, 0, 0)),
            pl.BlockSpec((3, 128), lambda i: (0, 0)),
        ],
        out_specs=[pl.BlockSpec((EB, 64), lambda i: (i, 0))] * 3,
        out_shape=[shp, shp, shp],
    )(ef_pad, m_stack, c_stack)


def _sprep_body(ssum_ref, cnt_ref, wa_ref, wb_ref, wc_ref, bias_ref,
                s_ref, s2s_ref, dsup_ref, s2mg_ref, k_head):
    s = ssum_ref[...] / jnp.maximum(cnt_ref[...], 1.0)
    s_ref[...] = s
    s2s_ref[...] = jnp.tanh(_DOT(s, wa_ref[...]) + bias_ref[0:1, :])
    s2mg_ref[...] = jnp.tanh(_DOT(s, wc_ref[...]) + bias_ref[1:2, :])
    for h in range(k_head):
        dsup_ref[h * 16:(h + 1) * 16, :] = jnp.tanh(
            _DOT(s, wb_ref[h]) + bias_ref[2 + h:3 + h, :])


def _sprep(ssum, cnt, wa, wb_stack, wc, bias, k_head):
    kk = k_head * 16
    return pl.pallas_call(
        functools.partial(_sprep_body, k_head=k_head),
        grid=(1,),
        in_specs=[
            pl.BlockSpec((16, 128), lambda i: (0, 0)),
            pl.BlockSpec((16, 128), lambda i: (0, 0)),
            pl.BlockSpec((128, 128), lambda i: (0, 0)),
            pl.BlockSpec((k_head, 128, 128), lambda i: (0, 0, 0)),
            pl.BlockSpec((128, 128), lambda i: (0, 0)),
            pl.BlockSpec((8, 128), lambda i: (0, 0)),
        ],
        out_specs=[
            pl.BlockSpec((16, 128), lambda i: (0, 0)),
            pl.BlockSpec((16, 128), lambda i: (0, 0)),
            pl.BlockSpec((kk, 128), lambda i: (0, 0)),
            pl.BlockSpec((16, 128), lambda i: (0, 0)),
        ],
        out_shape=[
            jax.ShapeDtypeStruct((16, 128), jnp.float32),
            jax.ShapeDtypeStruct((16, 128), jnp.float32),
            jax.ShapeDtypeStruct((kk, 128), jnp.float32),
            jax.ShapeDtypeStruct((16, 128), jnp.float32),
        ],
    )(ssum, cnt, wa, wb_stack, wc, bias)


def _p1_body(v_ref, oh_ref, wa_ref, cvec_ref, dsup_ref, bias_ref,
             amax_ref, k_head):
    v = v_ref[...]
    oh = oh_ref[...]
    @pl.when(pl.program_id(0) == 0)
    def _():
        amax_ref[...] = jnp.full_like(amax_ref, -1e30)
    cols = []
    for h in range(k_head):
        dn = jnp.tanh(_DOT(v, wa_ref[h]) + bias_ref[h:h + 1, :])
        dsn = _DOT(oh, dsup_ref[h * 16:(h + 1) * 16, :])
        a = jnp.sum(dn * dsn * cvec_ref[h:h + 1, :], axis=1, keepdims=True) \
            + bias_ref[4 + h:5 + h, 0:1]
        masked = jnp.where(oh > 0.0, a, -1e30)
        cols.append(jnp.max(masked, axis=0, keepdims=True))
    block = jnp.concatenate(cols + [jnp.full((8 - k_head, 16), -1e30)], axis=0)
    block = jnp.pad(block, ((0, 0), (0, 112)), constant_values=-1e30)
    amax_ref[...] = jnp.maximum(amax_ref[...], block)


def _p2_body(v_ref, oh_ref, wa_ref, cvec_ref, dsup_ref, wd_ref, amax_ref,
             bias_ref, num_ref, den_ref, k_head):
    v = v_ref[...]
    oh = oh_ref[...]
    @pl.when(pl.program_id(0) == 0)
    def _():
        num_ref[...] = jnp.zeros_like(num_ref)
        den_ref[...] = jnp.zeros_like(den_ref)
    den_cols = []
    for h in range(k_head):
        dn = jnp.tanh(_DOT(v, wa_ref[h]) + bias_ref[h:h + 1, :])
        dsn = _DOT(oh, dsup_ref[h * 16:(h + 1) * 16, :])
        a = jnp.sum(dn * dsn * cvec_ref[h:h + 1, :], axis=1, keepdims=True) \
            + bias_ref[4 + h:5 + h, 0:1]
        amax_n = _DOT(oh, amax_ref[h, 0:16])[:, None]
        ex = jnp.exp(a - amax_n)
        lind = _DOT(v, wd_ref[h]) + bias_ref[8 + h:9 + h, :]
        num_ref[h * 16:(h + 1) * 16, :] += _dotT(oh, ex * lind)
        den_cols.append(_dotT(oh, ex))
    den = jnp.concatenate(
        den_cols + [jnp.zeros((16, 128 - k_head), jnp.float32)], axis=1)
    den_ref[...] += den


def _helper_pass(v, oh, wa_stack, cvec, dsup, wd_stack, bias, k_head):
    grid = N // NB
    kk = k_head * 16
    amax = pl.pallas_call(
        functools.partial(_p1_body, k_head=k_head),
        grid=(grid,),
        in_specs=[
            pl.BlockSpec((NB, 128), lambda i: (i, 0)),
            pl.BlockSpec((NB, 16), lambda i: (i, 0)),
            pl.BlockSpec((k_head, 128, 128), lambda i: (0, 0, 0)),
            pl.BlockSpec((8, 128), lambda i: (0, 0)),
            pl.BlockSpec((kk, 128), lambda i: (0, 0)),
            pl.BlockSpec((16, 128), lambda i: (0, 0)),
        ],
        out_specs=pl.BlockSpec((8, 128), lambda i: (0, 0)),
        out_shape=jax.ShapeDtypeStruct((8, 128), jnp.float32),
    )(v, oh, wa_stack, cvec, dsup, bias)
    num, den = pl.pallas_call(
        functools.partial(_p2_body, k_head=k_head),
        grid=(grid,),
        in_specs=[
            pl.BlockSpec((NB, 128), lambda i: (i, 0)),
            pl.BlockSpec((NB, 16), lambda i: (i, 0)),
            pl.BlockSpec((k_head, 128, 128), lambda i: (0, 0, 0)),
            pl.BlockSpec((8, 128), lambda i: (0, 0)),
            pl.BlockSpec((kk, 128), lambda i: (0, 0)),
            pl.BlockSpec((k_head, 128, 128), lambda i: (0, 0, 0)),
            pl.BlockSpec((8, 128), lambda i: (0, 0)),
            pl.BlockSpec((16, 128), lambda i: (0, 0)),
        ],
        out_specs=[
            pl.BlockSpec((kk, 128), lambda i: (0, 0)),
            pl.BlockSpec((16, 128), lambda i: (0, 0)),
        ],
        out_shape=[
            jax.ShapeDtypeStruct((kk, 128), jnp.float32),
            jax.ShapeDtypeStruct((16, 128), jnp.float32),
        ],
    )(v, oh, wa_stack, cvec, dsup, wd_stack, amax, bias)
    return num, den


def _gates_body(v_ref, sve0_ref, sve1_ref, oh_ref, s2mg_ref,
                wes_ref, wev_ref, wa_ref, wb_ref, wih_ref, whh_ref,
                bias_ref, vv_ref):
    v = v_ref[...]
    oh = oh_ref[...]
    sve = sve0_ref[0] + sve1_ref[0]
    m2m = _lrelu(_DOT(sve, wes_ref[...]) + _DOT(v, wev_ref[...])
                 + bias_ref[0:1, :])
    s2m = _DOT(oh, s2mg_ref[...])
    z = jax.nn.sigmoid(_DOT(m2m, wa_ref[...]) + _DOT(s2m, wb_ref[...])
                       + bias_ref[1:2, :])
    h = z * s2m + (1.0 - z) * m2m
    bih = jnp.concatenate([bias_ref[2:3, :], bias_ref[3:4, :],
                           bias_ref[4:5, :]], axis=1)
    bhh = jnp.concatenate([bias_ref[5:6, :], bias_ref[6:7, :],
                           bias_ref[7:8, :]], axis=1)
    gi = _DOT(v, wih_ref[...]) + bih
    gh = _DOT(h, whh_ref[...]) + bhh
    r = jax.nn.sigmoid(gi[:, 0:128] + gh[:, 0:128])
    z2 = jax.nn.sigmoid(gi[:, 128:256] + gh[:, 128:256])
    n = jnp.tanh(gi[:, 256:384] + r * gh[:, 256:384])
    vv_ref[...] = (1.0 - z2) * n + z2 * h


def _gates(v, parts, oh, s2mg, wes, wev, wa, wb, wih, whh, bias):
    grid = N // NB
    return pl.pallas_call(
        _gates_body,
        grid=(grid,),
        in_specs=[
            pl.BlockSpec((NB, 128), lambda i: (i, 0)),
            pl.BlockSpec((1, NB, 128), lambda i: (0, i, 0)),
            pl.BlockSpec((1, NB, 128), lambda i: (1, i, 0)),
            pl.BlockSpec((NB, 16), lambda i: (i, 0)),
            pl.BlockSpec((16, 128), lambda i: (0, 0)),
            pl.BlockSpec((128, 128), lambda i: (0, 0)),
            pl.BlockSpec((128, 128), lambda i: (0, 0)),
            pl.BlockSpec((128, 128), lambda i: (0, 0)),
            pl.BlockSpec((128, 128), lambda i: (0, 0)),
            pl.BlockSpec((128, 384), lambda i: (0, 0)),
            pl.BlockSpec((128, 384), lambda i: (0, 0)),
            pl.BlockSpec((8, 128), lambda i: (0, 0)),
        ],
        out_specs=pl.BlockSpec((NB, 128), lambda i: (i, 0)),
        out_shape=jax.ShapeDtypeStruct((N, 128), jnp.float32),
    )(v, parts, parts, oh, s2mg, wes, wev, wa, wb, wih, whh, bias)


def _supdate_body(s_ref, s2s_ref, num_ref, den_ref, wm_ref, wa_ref, wb_ref,
                  wih_ref, whh_ref, bias_ref, ss_ref, k_head):
    s = s_ref[...]
    s2s = s2s_ref[...]
    houts = []
    for h in range(k_head):
        d = jnp.maximum(den_ref[:, h:h + 1], 1e-30)
        houts.append(num_ref[h * 16:(h + 1) * 16, :] / d)
    mcat = jnp.concatenate(houts, axis=1)
    m2s = jnp.tanh(_DOT(mcat, wm_ref[...]) + bias_ref[0:1, :])
    z = jax.nn.sigmoid(_DOT(s2s, wa_ref[...]) + _DOT(m2s, wb_ref[...])
                       + bias_ref[1:2, :])
    h = z * m2s + (1.0 - z) * s2s
    bih = jnp.concatenate([bias_ref[2:3, :], bias_ref[3:4, :],
                           bias_ref[4:5, :]], axis=1)
    bhh = jnp.concatenate([bias_ref[5:6, :], bias_ref[6:7, :],
                           bias_ref[7:8, :]], axis=1)
    gi = _DOT(s, wih_ref[...]) + bih
    gh = _DOT(h, whh_ref[...]) + bhh
    r = jax.nn.sigmoid(gi[:, 0:128] + gh[:, 0:128])
    z2 = jax.nn.sigmoid(gi[:, 128:256] + gh[:, 128:256])
    n = jnp.tanh(gi[:, 256:384] + r * gh[:, 256:384])
    ss_ref[...] = (1.0 - z2) * n + z2 * h


def _supdate(s, s2s, num, den, wm, wa, wb, wih, whh, bias, k_head):
    kk = k_head * 16
    return pl.pallas_call(
        functools.partial(_supdate_body, k_head=k_head),
        grid=(1,),
        in_specs=[
            pl.BlockSpec((16, 128), lambda i: (0, 0)),
            pl.BlockSpec((16, 128), lambda i: (0, 0)),
            pl.BlockSpec((kk, 128), lambda i: (0, 0)),
            pl.BlockSpec((16, 128), lambda i: (0, 0)),
            pl.BlockSpec((k_head * 128, 128), lambda i: (0, 0)),
            pl.BlockSpec((128, 128), lambda i: (0, 0)),
            pl.BlockSpec((128, 128), lambda i: (0, 0)),
            pl.BlockSpec((128, 384), lambda i: (0, 0)),
            pl.BlockSpec((128, 384), lambda i: (0, 0)),
            pl.BlockSpec((8, 128), lambda i: (0, 0)),
        ],
        out_specs=pl.BlockSpec((16, 128), lambda i: (0, 0)),
        out_shape=jax.ShapeDtypeStruct((16, 128), jnp.float32),
    )(s, s2s, num, den, wm, wa, wb, wih, whh, bias)


def _prep_weights(params):
    P = {}
    p0 = params["a_init"]
    w0 = jnp.pad(p0["W"].T, ((0, 2), (0, 0)))
    w1 = params["a_init1"]["W"].T
    w2 = params["a_init2"]["W"].T
    bias = jnp.zeros((8, 128), jnp.float32)
    bias = bias.at[0].set(p0["b"])
    bias = bias.at[1, :64].set(params["a_init1"]["b"])
    bias = bias.at[2].set(params["a_init2"]["b"])
    P["init"] = (w0, w1, w2, bias,
                 params["norml"]["g"][None, :], params["norml"]["b"][None, :])

    wb = params["b_init"]
    base = (jnp.arange(4)[:, None] * 32 + jnp.arange(16)[None, :])
    perm = jnp.concatenate([base.reshape(-1), (base + 16).reshape(-1)])
    m_stack, c_stack = [], []
    for name in ("conv1", "conv2", "conv3"):
        K = params[name]["K"]
        m = (K["W"] @ wb["W"]).T
        m_stack.append(jnp.pad(m[:, perm], ((0, 4), (0, 0))))
        c_stack.append((wb["b"] @ K["W"].T + K["b"])[perm])
    P["ke"] = (jnp.stack(m_stack), jnp.stack(c_stack))

    for name in ("conv1", "conv2", "conv3"):
        p = params[name]
        k_head = len(p["helpers"])
        L = {}
        sb = jnp.zeros((8, 128), jnp.float32)
        sb = sb.at[0].set(p["A"]["b"])
        sb = sb.at[1].set(p["C"]["b"])
        for h, hp in enumerate(p["helpers"]):
            sb = sb.at[2 + h].set(hp["B"]["b"])
        L["sprep"] = (p["A"]["W"].T,
                      jnp.stack([hp["B"]["W"].T for hp in p["helpers"]]),
                      p["C"]["W"].T, sb, k_head)

        hb = jnp.zeros((16, 128), jnp.float32)
        cvec = jnp.zeros((8, 128), jnp.float32)
        for h, hp in enumerate(p["helpers"]):
            hb = hb.at[h].set(hp["A"]["b"])
            hb = hb.at[4 + h].set(jnp.broadcast_to(hp["C"]["b"], (128,)))
            hb = hb.at[8 + h].set(hp["D"]["b"])
            cvec = cvec.at[h].set(hp["C"]["W"][0])
        L["helper"] = (jnp.stack([hp["A"]["W"].T for hp in p["helpers"]]),
                       cvec,
                       jnp.stack([hp["D"]["W"].T for hp in p["helpers"]]),
                       hb, k_head)

        gm = p["gm"]
        gb = jnp.zeros((8, 128), jnp.float32)
        gb = gb.at[0].set(p["E"]["b"])
        gb = gb.at[1].set(gm["A"]["b"] + gm["B"]["b"])
        for t in range(3):
            gb = gb.at[2 + t].set(gm["bih"][t * 128:(t + 1) * 128])
            gb = gb.at[5 + t].set(gm["bhh"][t * 128:(t + 1) * 128])
        we = p["E"]["W"].T
        L["gates"] = (we[0:128], we[128:256], gm["A"]["W"].T, gm["B"]["W"].T,
                      gm["Wih"].T, gm["Whh"].T, gb)

        gs = p["gs"]
        ub = jnp.zeros((8, 128), jnp.float32)
        ub = ub.at[0].set(p["B"]["b"])
        ub = ub.at[1].set(gs["A"]["b"] + gs["B"]["b"])
        for t in range(3):
            ub = ub.at[2 + t].set(gs["bih"][t * 128:(t + 1) * 128])
            ub = ub.at[5 + t].set(gs["bhh"][t * 128:(t + 1) * 128])
        L["supdate"] = (p["B"]["W"].T, gs["A"]["W"].T, gs["B"]["W"].T,
                        gs["Wih"].T, gs["Whh"].T, ub, k_head)
        P[name] = L
    return P


def kernel(x, edge_feat, edge_index, graph_ids, params):
    src_pad = jnp.pad(edge_index[0], (0, E_PAD - E)).reshape(NCHUNKS, CH)
    dst_pad = jnp.pad(edge_index[1], (0, E_PAD - E)).reshape(NCHUNKS, CH)
    ef_pad = jnp.pad(edge_feat, ((0, E_PAD - E), (0, 4)))
    x_pad = jnp.pad(x, ((0, 0), (0, 2)))
    oh = (graph_ids[:, None] == jnp.arange(16)[None, :]).astype(jnp.float32)

    P = _prep_weights(params)
    v, ssum, cnt = _node_init(x_pad, oh, *P["init"])
    kes = _ke_all(ef_pad, *P["ke"])

    sraw, craw = ssum, cnt
    for li, name in enumerate(("conv1", "conv2", "conv3")):
        L = P[name]
        s, s2s, dsup, s2mg = _sprep(sraw, craw, *L["sprep"])
        wa_stack, cvec, wd_stack, hbias, k_head = L["helper"]
        num, den = _helper_pass(v, oh, wa_stack, cvec, dsup, wd_stack,
                                hbias, k_head)
        parts = _edge_aggregate(v, kes[li], src_pad, dst_pad)
        vv = _gates(v, parts, oh, s2mg, *L["gates"])
        ss = _supdate(s, s2s, num, den, *L["supdate"])
        v, sraw, craw = vv, ss, jnp.ones((16, 128), jnp.float32)
    return v


# speedup vs baseline: 2.1674x; 1.4640x over previous
import functools

import jax
import jax.numpy as jnp
from jax import lax
from jax.experimental import pallas as pl
from jax.experimental.pallas import tpu as pltpu
from jax.experimental.pallas import tpu_sc as plsc

N = 10000
E = 160000
B = 16
HD = 128

NC = 2
NS = 16
NW = NC * NS

CH = 96
CPW0 = 94
CPW1 = 14
E_PAD = NS * (CPW0 + CPW1) * CH
NCHUNKS = E_PAD // CH
N_ACC = 10112
ZPW = N_ACC // NS

NB = 1000
EB = 2048

_DOT = functools.partial(jnp.dot, precision=lax.Precision.DEFAULT)


def _dotT(a, b):
    return lax.dot_general(a, b, (((0,), (0,)), ((), ())),
                           precision=lax.Precision.DEFAULT)


def _lrelu(x):
    return jnp.maximum(x, 0.1 * x)


def _edge_sc_body(v_hbm, ke_hbm, src_hbm, dst_hbm, out_hbm,
                  isrc, idst, rows0, kev0, rows1, kev1, sve_sh,
                  isem0, isem1, lsem0, lsem1, ssem0, ssem1):
    c = lax.axis_index("c")
    s = lax.axis_index("s")
    wid = c * NS + s

    def zrow(r, _):
        for j in range(8):
            rows0[r, pl.ds(j * 16, 16)] = jnp.zeros((16,), jnp.float32)
        return 0
    lax.fori_loop(0, CH, zrow, 0)
    zrem = ZPW - (ZPW // CH) * CH
    for j in range(ZPW // CH):
        pltpu.sync_copy(rows0, sve_sh.at[pl.ds(s * ZPW + j * CH, CH)])
    if zrem:
        pltpu.sync_copy(rows0.at[pl.ds(0, zrem)],
                        sve_sh.at[pl.ds(s * ZPW + (ZPW // CH) * CH, zrem)])
    plsc.subcore_barrier()

    cpw = lax.select(c == 0, jnp.int32(CPW0), jnp.int32(CPW1))
    cbase = c * NS * CPW0 + s * cpw
    isems = (isem0, isem1)

    def issue_src(g, b):
        pltpu.async_copy(src_hbm.at[cbase + g], isrc.at[b], isems[b])

    def wait_src(g, b):
        pltpu.make_async_copy(src_hbm.at[cbase + g], isrc.at[b],
                              isems[b]).wait()

    def issue_dst(g, b):
        pltpu.async_copy(dst_hbm.at[cbase + g], idst.at[b], isems[b])

    def wait_dst(g, b):
        pltpu.make_async_copy(dst_hbm.at[cbase + g], idst.at[b],
                              isems[b]).wait()

    def issue_data(g, b, rows, kev, lsem):
        pltpu.async_copy(v_hbm.at[isrc.at[b]], rows, lsem)
        pltpu.async_copy(ke_hbm.at[pl.ds((cbase + g) * CH, CH)], kev, lsem)

    def wait_data(g, b, rows, kev, lsem):
        pltpu.make_async_copy(v_hbm.at[isrc.at[b]], rows, lsem).wait()
        pltpu.make_async_copy(
            ke_hbm.at[pl.ds((cbase + g) * CH, CH)], kev, lsem).wait()

    npair = cpw // 2

    def issue_scatter(b, rows, ssem):
        pltpu.async_copy(rows, sve_sh.at[idst.at[b]], ssem, add=True)

    def wait_scatter(b, rows, ssem):
        pltpu.make_async_copy(rows, sve_sh.at[idst.at[b]], ssem).wait()

    def compute(rows, kev):
        @plsc.parallel_loop(0, CH, unroll=2)
        def _(r):
            for j in range(8):
                t = rows[r, pl.ds(j * 16, 16)] * kev[r, pl.ds(j * 16, 16)]
                rows[r, pl.ds(j * 16, 16)] = jnp.maximum(t, t * 0.1)

    issue_src(0, 0)
    wait_src(0, 0)
    issue_dst(0, 0)
    issue_data(0, 0, rows0, kev0, lsem0)
    issue_src(1, 1)

    def pair(i, _):
        g0 = 2 * i
        g1 = g0 + 1
        last = npair - 1

        @pl.when(i > 0)
        def _():
            wait_scatter(1, rows1, ssem1)
        wait_src(g1, 1)
        issue_data(g1, 1, rows1, kev1, lsem1)
        issue_dst(g1, 1)
        wait_data(g0, 0, rows0, kev0, lsem0)
        wait_dst(g0, 0)

        @pl.when(i < last)
        def _():
            issue_src(g0 + 2, 0)
        compute(rows0, kev0)
        issue_scatter(0, rows0, ssem0)

        wait_scatter(0, rows0, ssem0)

        @pl.when(i < last)
        def _():
            wait_src(g0 + 2, 0)
            issue_data(g0 + 2, 0, rows0, kev0, lsem0)
            issue_dst(g0 + 2, 0)
        wait_data(g1, 1, rows1, kev1, lsem1)
        wait_dst(g1, 1)

        @pl.when(i < last)
        def _():
            issue_src(g1 + 2, 1)
        compute(rows1, kev1)
        issue_scatter(1, rows1, ssem1)
        return 0

    lax.fori_loop(0, npair, pair, 0)
    wait_scatter(1, rows1, ssem1)
    plsc.subcore_barrier()

    pltpu.sync_copy(sve_sh.at[pl.ds(s * ZPW, ZPW)],
                    out_hbm.at[c, pl.ds(s * ZPW, ZPW)])


def _edge_aggregate(v, ke_pad, src_3d, dst_3d):
    mesh = plsc.VectorSubcoreMesh(core_axis_name="c", subcore_axis_name="s")
    return pl.kernel(
        _edge_sc_body,
        out_type=jax.ShapeDtypeStruct((NC, N_ACC, 128), jnp.float32),
        mesh=mesh,
        scratch_types=[
            pltpu.VMEM((2, CH), jnp.int32),
            pltpu.VMEM((2, CH), jnp.int32),
            pltpu.VMEM((CH, 128), jnp.float32),
            pltpu.VMEM((CH, 128), jnp.float32),
            pltpu.VMEM((CH, 128), jnp.float32),
            pltpu.VMEM((CH, 128), jnp.float32),
            pltpu.VMEM_SHARED((N_ACC, 128), jnp.float32),
            pltpu.SemaphoreType.DMA,
            pltpu.SemaphoreType.DMA,
            pltpu.SemaphoreType.DMA,
            pltpu.SemaphoreType.DMA,
            pltpu.SemaphoreType.DMA,
            pltpu.SemaphoreType.DMA,
        ],
    )(v, ke_pad, src_3d, dst_3d)


def _init_body(x_ref, oh_ref, w0_ref, w1_ref, w2_ref, bias_ref, g_ref, b_ref,
               v_ref, ssum_ref, cnt_ref):
    x = x_ref[...]
    oh = oh_ref[...]
    v = _DOT(x, w0_ref[...]) + bias_ref[0:1, :]
    h = jnp.maximum(_DOT(v, w1_ref[...]) + bias_ref[1:2, :64], 0.0)
    v = _DOT(h, w2_ref[...]) + bias_ref[2:3, :]
    m = jnp.mean(v, -1, keepdims=True)
    var = jnp.mean((v - m) ** 2, -1, keepdims=True)
    v = (v - m) / jnp.sqrt(var + 1e-5) * g_ref[0:1, :] + b_ref[0:1, :]
    v_ref[...] = v

    @pl.when(pl.program_id(0) == 0)
    def _():
        ssum_ref[...] = jnp.zeros_like(ssum_ref)
        cnt_ref[...] = jnp.zeros_like(cnt_ref)
    ssum_ref[...] += _dotT(oh, v)
    cnt_ref[...] += _dotT(oh, jnp.ones_like(v))


def _node_init(x_pad, oh, w0, w1, w2, bias, g, b):
    grid = N // NB
    return pl.pallas_call(
        _init_body,
        grid=(grid,),
        in_specs=[
            pl.BlockSpec((NB, 384), lambda i: (i, 0)),
            pl.BlockSpec((NB, 16), lambda i: (i, 0)),
            pl.BlockSpec((384, 128), lambda i: (0, 0)),
            pl.BlockSpec((128, 64), lambda i: (0, 0)),
            pl.BlockSpec((64, 128), lambda i: (0, 0)),
            pl.BlockSpec((8, 128), lambda i: (0, 0)),
            pl.BlockSpec((1, 128), lambda i: (0, 0)),
            pl.BlockSpec((1, 128), lambda i: (0, 0)),
        ],
        out_specs=[
            pl.BlockSpec((NB, 128), lambda i: (i, 0)),
            pl.BlockSpec((16, 128), lambda i: (0, 0)),
            pl.BlockSpec((16, 128), lambda i: (0, 0)),
        ],
        out_shape=[
            jax.ShapeDtypeStruct((N, 128), jnp.float32),
            jax.ShapeDtypeStruct((16, 128), jnp.float32),
            jax.ShapeDtypeStruct((16, 128), jnp.float32),
        ],
    )(x_pad, oh, w0, w1, w2, bias, g, b)


def _ke_body(ef_ref, m_ref, c_ref, ke1_ref, ke2_ref, ke3_ref):
    ef = ef_ref[...]
    rid = pl.program_id(0) * EB + lax.broadcasted_iota(jnp.int32, (EB, 1), 0)
    valid = rid < E
    for l, out in enumerate((ke1_ref, ke2_ref, ke3_ref)):
        ke = _DOT(ef, m_ref[l]) + c_ref[l:l + 1, :]
        out[...] = jnp.where(valid, ke, 0.0)


def _ke_all(ef_pad, m_stack, c_stack):
    grid = E_PAD // EB
    shp = jax.ShapeDtypeStruct((E_PAD, 128), jnp.float32)
    return pl.pallas_call(
        _ke_body,
        grid=(grid,),
        in_specs=[
            pl.BlockSpec((EB, 16), lambda i: (i, 0)),
            pl.BlockSpec((3, 16, 128), lambda i: (0, 0, 0)),
            pl.BlockSpec((3, 128), lambda i: (0, 0)),
        ],
        out_specs=[pl.BlockSpec((EB, 128), lambda i: (i, 0))] * 3,
        out_shape=[shp, shp, shp],
    )(ef_pad, m_stack, c_stack)


def _sprep_body(ssum_ref, cnt_ref, wa_ref, wb_ref, wc_ref, bias_ref,
                s_ref, s2s_ref, dsup_ref, s2mg_ref, k_head):
    s = ssum_ref[...] / jnp.maximum(cnt_ref[...], 1.0)
    s_ref[...] = s
    s2s_ref[...] = jnp.tanh(_DOT(s, wa_ref[...]) + bias_ref[0:1, :])
    s2mg_ref[...] = jnp.tanh(_DOT(s, wc_ref[...]) + bias_ref[1:2, :])
    for h in range(k_head):
        dsup_ref[h * 16:(h + 1) * 16, :] = jnp.tanh(
            _DOT(s, wb_ref[h]) + bias_ref[2 + h:3 + h, :])


def _sprep(ssum, cnt, wa, wb_stack, wc, bias, k_head):
    kk = k_head * 16
    return pl.pallas_call(
        functools.partial(_sprep_body, k_head=k_head),
        grid=(1,),
        in_specs=[
            pl.BlockSpec((16, 128), lambda i: (0, 0)),
            pl.BlockSpec((16, 128), lambda i: (0, 0)),
            pl.BlockSpec((128, 128), lambda i: (0, 0)),
            pl.BlockSpec((k_head, 128, 128), lambda i: (0, 0, 0)),
            pl.BlockSpec((128, 128), lambda i: (0, 0)),
            pl.BlockSpec((8, 128), lambda i: (0, 0)),
        ],
        out_specs=[
            pl.BlockSpec((16, 128), lambda i: (0, 0)),
            pl.BlockSpec((16, 128), lambda i: (0, 0)),
            pl.BlockSpec((kk, 128), lambda i: (0, 0)),
            pl.BlockSpec((16, 128), lambda i: (0, 0)),
        ],
        out_shape=[
            jax.ShapeDtypeStruct((16, 128), jnp.float32),
            jax.ShapeDtypeStruct((16, 128), jnp.float32),
            jax.ShapeDtypeStruct((kk, 128), jnp.float32),
            jax.ShapeDtypeStruct((16, 128), jnp.float32),
        ],
    )(ssum, cnt, wa, wb_stack, wc, bias)


def _p1_body(v_ref, oh_ref, wa_ref, cvec_ref, dsup_ref, bias_ref,
             amax_ref, k_head):
    v = v_ref[...]
    oh = oh_ref[...]
    @pl.when(pl.program_id(0) == 0)
    def _():
        amax_ref[...] = jnp.full_like(amax_ref, -1e30)
    cols = []
    for h in range(k_head):
        dn = jnp.tanh(_DOT(v, wa_ref[h]) + bias_ref[h:h + 1, :])
        dsn = _DOT(oh, dsup_ref[h * 16:(h + 1) * 16, :])
        a = jnp.sum(dn * dsn * cvec_ref[h:h + 1, :], axis=1, keepdims=True) \
            + bias_ref[4 + h:5 + h, 0:1]
        masked = jnp.where(oh > 0.0, a, -1e30)
        cols.append(jnp.max(masked, axis=0, keepdims=True))
    block = jnp.concatenate(cols + [jnp.full((8 - k_head, 16), -1e30)], axis=0)
    block = jnp.pad(block, ((0, 0), (0, 112)), constant_values=-1e30)
    amax_ref[...] = jnp.maximum(amax_ref[...], block)


def _p2_body(v_ref, oh_ref, wa_ref, cvec_ref, dsup_ref, wd_ref, amax_ref,
             bias_ref, num_ref, den_ref, k_head):
    v = v_ref[...]
    oh = oh_ref[...]
    @pl.when(pl.program_id(0) == 0)
    def _():
        num_ref[...] = jnp.zeros_like(num_ref)
        den_ref[...] = jnp.zeros_like(den_ref)
    den_cols = []
    for h in range(k_head):
        dn = jnp.tanh(_DOT(v, wa_ref[h]) + bias_ref[h:h + 1, :])
        dsn = _DOT(oh, dsup_ref[h * 16:(h + 1) * 16, :])
        a = jnp.sum(dn * dsn * cvec_ref[h:h + 1, :], axis=1, keepdims=True) \
            + bias_ref[4 + h:5 + h, 0:1]
        amax_n = _DOT(oh, amax_ref[h, 0:16])[:, None]
        ex = jnp.exp(a - amax_n)
        lind = _DOT(v, wd_ref[h]) + bias_ref[8 + h:9 + h, :]
        num_ref[h * 16:(h + 1) * 16, :] += _dotT(oh, ex * lind)
        den_cols.append(_dotT(oh, ex))
    den = jnp.concatenate(
        den_cols + [jnp.zeros((16, 128 - k_head), jnp.float32)], axis=1)
    den_ref[...] += den


def _helper_pass(v, oh, wa_stack, cvec, dsup, wd_stack, bias, k_head):
    grid = N // NB
    kk = k_head * 16
    amax = pl.pallas_call(
        functools.partial(_p1_body, k_head=k_head),
        grid=(grid,),
        in_specs=[
            pl.BlockSpec((NB, 128), lambda i: (i, 0)),
            pl.BlockSpec((NB, 16), lambda i: (i, 0)),
            pl.BlockSpec((k_head, 128, 128), lambda i: (0, 0, 0)),
            pl.BlockSpec((8, 128), lambda i: (0, 0)),
            pl.BlockSpec((kk, 128), lambda i: (0, 0)),
            pl.BlockSpec((16, 128), lambda i: (0, 0)),
        ],
        out_specs=pl.BlockSpec((8, 128), lambda i: (0, 0)),
        out_shape=jax.ShapeDtypeStruct((8, 128), jnp.float32),
    )(v, oh, wa_stack, cvec, dsup, bias)
    num, den = pl.pallas_call(
        functools.partial(_p2_body, k_head=k_head),
        grid=(grid,),
        in_specs=[
            pl.BlockSpec((NB, 128), lambda i: (i, 0)),
            pl.BlockSpec((NB, 16), lambda i: (i, 0)),
            pl.BlockSpec((k_head, 128, 128), lambda i: (0, 0, 0)),
            pl.BlockSpec((8, 128), lambda i: (0, 0)),
            pl.BlockSpec((kk, 128), lambda i: (0, 0)),
            pl.BlockSpec((k_head, 128, 128), lambda i: (0, 0, 0)),
            pl.BlockSpec((8, 128), lambda i: (0, 0)),
            pl.BlockSpec((16, 128), lambda i: (0, 0)),
        ],
        out_specs=[
            pl.BlockSpec((kk, 128), lambda i: (0, 0)),
            pl.BlockSpec((16, 128), lambda i: (0, 0)),
        ],
        out_shape=[
            jax.ShapeDtypeStruct((kk, 128), jnp.float32),
            jax.ShapeDtypeStruct((16, 128), jnp.float32),
        ],
    )(v, oh, wa_stack, cvec, dsup, wd_stack, amax, bias)
    return num, den


def _gates_body(v_ref, sve0_ref, sve1_ref, oh_ref, s2mg_ref,
                wes_ref, wev_ref, wa_ref, wb_ref, wih_ref, whh_ref,
                bias_ref, vv_ref):
    v = v_ref[...]
    oh = oh_ref[...]
    sve = sve0_ref[0] + sve1_ref[0]
    m2m = _lrelu(_DOT(sve, wes_ref[...]) + _DOT(v, wev_ref[...])
                 + bias_ref[0:1, :])
    s2m = _DOT(oh, s2mg_ref[...])
    z = jax.nn.sigmoid(_DOT(m2m, wa_ref[...]) + _DOT(s2m, wb_ref[...])
                       + bias_ref[1:2, :])
    h = z * s2m + (1.0 - z) * m2m
    bih = jnp.concatenate([bias_ref[2:3, :], bias_ref[3:4, :],
                           bias_ref[4:5, :]], axis=1)
    bhh = jnp.concatenate([bias_ref[5:6, :], bias_ref[6:7, :],
                           bias_ref[7:8, :]], axis=1)
    gi = _DOT(v, wih_ref[...]) + bih
    gh = _DOT(h, whh_ref[...]) + bhh
    r = jax.nn.sigmoid(gi[:, 0:128] + gh[:, 0:128])
    z2 = jax.nn.sigmoid(gi[:, 128:256] + gh[:, 128:256])
    n = jnp.tanh(gi[:, 256:384] + r * gh[:, 256:384])
    vv_ref[...] = (1.0 - z2) * n + z2 * h


def _gates(v, parts, oh, s2mg, wes, wev, wa, wb, wih, whh, bias):
    grid = N // NB
    return pl.pallas_call(
        _gates_body,
        grid=(grid,),
        in_specs=[
            pl.BlockSpec((NB, 128), lambda i: (i, 0)),
            pl.BlockSpec((1, NB, 128), lambda i: (0, i, 0)),
            pl.BlockSpec((1, NB, 128), lambda i: (1, i, 0)),
            pl.BlockSpec((NB, 16), lambda i: (i, 0)),
            pl.BlockSpec((16, 128), lambda i: (0, 0)),
            pl.BlockSpec((128, 128), lambda i: (0, 0)),
            pl.BlockSpec((128, 128), lambda i: (0, 0)),
            pl.BlockSpec((128, 128), lambda i: (0, 0)),
            pl.BlockSpec((128, 128), lambda i: (0, 0)),
            pl.BlockSpec((128, 384), lambda i: (0, 0)),
            pl.BlockSpec((128, 384), lambda i: (0, 0)),
            pl.BlockSpec((8, 128), lambda i: (0, 0)),
        ],
        out_specs=pl.BlockSpec((NB, 128), lambda i: (i, 0)),
        out_shape=jax.ShapeDtypeStruct((N, 128), jnp.float32),
    )(v, parts, parts, oh, s2mg, wes, wev, wa, wb, wih, whh, bias)


def _supdate_body(s_ref, s2s_ref, num_ref, den_ref, wm_ref, wa_ref, wb_ref,
                  wih_ref, whh_ref, bias_ref, ss_ref, k_head):
    s = s_ref[...]
    s2s = s2s_ref[...]
    houts = []
    for h in range(k_head):
        d = jnp.maximum(den_ref[:, h:h + 1], 1e-30)
        houts.append(num_ref[h * 16:(h + 1) * 16, :] / d)
    mcat = jnp.concatenate(houts, axis=1)
    m2s = jnp.tanh(_DOT(mcat, wm_ref[...]) + bias_ref[0:1, :])
    z = jax.nn.sigmoid(_DOT(s2s, wa_ref[...]) + _DOT(m2s, wb_ref[...])
                       + bias_ref[1:2, :])
    h = z * m2s + (1.0 - z) * s2s
    bih = jnp.concatenate([bias_ref[2:3, :], bias_ref[3:4, :],
                           bias_ref[4:5, :]], axis=1)
    bhh = jnp.concatenate([bias_ref[5:6, :], bias_ref[6:7, :],
                           bias_ref[7:8, :]], axis=1)
    gi = _DOT(s, wih_ref[...]) + bih
    gh = _DOT(h, whh_ref[...]) + bhh
    r = jax.nn.sigmoid(gi[:, 0:128] + gh[:, 0:128])
    z2 = jax.nn.sigmoid(gi[:, 128:256] + gh[:, 128:256])
    n = jnp.tanh(gi[:, 256:384] + r * gh[:, 256:384])
    ss_ref[...] = (1.0 - z2) * n + z2 * h


def _supdate(s, s2s, num, den, wm, wa, wb, wih, whh, bias, k_head):
    kk = k_head * 16
    return pl.pallas_call(
        functools.partial(_supdate_body, k_head=k_head),
        grid=(1,),
        in_specs=[
            pl.BlockSpec((16, 128), lambda i: (0, 0)),
            pl.BlockSpec((16, 128), lambda i: (0, 0)),
            pl.BlockSpec((kk, 128), lambda i: (0, 0)),
            pl.BlockSpec((16, 128), lambda i: (0, 0)),
            pl.BlockSpec((k_head * 128, 128), lambda i: (0, 0)),
            pl.BlockSpec((128, 128), lambda i: (0, 0)),
            pl.BlockSpec((128, 128), lambda i: (0, 0)),
            pl.BlockSpec((128, 384), lambda i: (0, 0)),
            pl.BlockSpec((128, 384), lambda i: (0, 0)),
            pl.BlockSpec((8, 128), lambda i: (0, 0)),
        ],
        out_specs=pl.BlockSpec((16, 128), lambda i: (0, 0)),
        out_shape=jax.ShapeDtypeStruct((16, 128), jnp.float32),
    )(s, s2s, num, den, wm, wa, wb, wih, whh, bias)


def _prep_weights(params):
    P = {}
    p0 = params["a_init"]
    w0 = jnp.pad(p0["W"].T, ((0, 2), (0, 0)))
    w1 = params["a_init1"]["W"].T
    w2 = params["a_init2"]["W"].T
    bias = jnp.zeros((8, 128), jnp.float32)
    bias = bias.at[0].set(p0["b"])
    bias = bias.at[1, :64].set(params["a_init1"]["b"])
    bias = bias.at[2].set(params["a_init2"]["b"])
    P["init"] = (w0, w1, w2, bias,
                 params["norml"]["g"][None, :], params["norml"]["b"][None, :])

    wb = params["b_init"]
    m_stack, c_stack = [], []
    for name in ("conv1", "conv2", "conv3"):
        K = params[name]["K"]
        m = (K["W"] @ wb["W"]).T
        m_stack.append(jnp.pad(m, ((0, 4), (0, 0))))
        c_stack.append(wb["b"] @ K["W"].T + K["b"])
    P["ke"] = (jnp.stack(m_stack), jnp.stack(c_stack))

    for name in ("conv1", "conv2", "conv3"):
        p = params[name]
        k_head = len(p["helpers"])
        L = {}
        sb = jnp.zeros((8, 128), jnp.float32)
        sb = sb.at[0].set(p["A"]["b"])
        sb = sb.at[1].set(p["C"]["b"])
        for h, hp in enumerate(p["helpers"]):
            sb = sb.at[2 + h].set(hp["B"]["b"])
        L["sprep"] = (p["A"]["W"].T,
                      jnp.stack([hp["B"]["W"].T for hp in p["helpers"]]),
                      p["C"]["W"].T, sb, k_head)

        hb = jnp.zeros((16, 128), jnp.float32)
        cvec = jnp.zeros((8, 128), jnp.float32)
        for h, hp in enumerate(p["helpers"]):
            hb = hb.at[h].set(hp["A"]["b"])
            hb = hb.at[4 + h].set(jnp.broadcast_to(hp["C"]["b"], (128,)))
            hb = hb.at[8 + h].set(hp["D"]["b"])
            cvec = cvec.at[h].set(hp["C"]["W"][0])
        L["helper"] = (jnp.stack([hp["A"]["W"].T for hp in p["helpers"]]),
                       cvec,
                       jnp.stack([hp["D"]["W"].T for hp in p["helpers"]]),
                       hb, k_head)

        gm = p["gm"]
        gb = jnp.zeros((8, 128), jnp.float32)
        gb = gb.at[0].set(p["E"]["b"])
        gb = gb.at[1].set(gm["A"]["b"] + gm["B"]["b"])
        for t in range(3):
            gb = gb.at[2 + t].set(gm["bih"][t * 128:(t + 1) * 128])
            gb = gb.at[5 + t].set(gm["bhh"][t * 128:(t + 1) * 128])
        we = p["E"]["W"].T
        L["gates"] = (we[0:128], we[128:256], gm["A"]["W"].T, gm["B"]["W"].T,
                      gm["Wih"].T, gm["Whh"].T, gb)

        gs = p["gs"]
        ub = jnp.zeros((8, 128), jnp.float32)
        ub = ub.at[0].set(p["B"]["b"])
        ub = ub.at[1].set(gs["A"]["b"] + gs["B"]["b"])
        for t in range(3):
            ub = ub.at[2 + t].set(gs["bih"][t * 128:(t + 1) * 128])
            ub = ub.at[5 + t].set(gs["bhh"][t * 128:(t + 1) * 128])
        L["supdate"] = (p["B"]["W"].T, gs["A"]["W"].T, gs["B"]["W"].T,
                        gs["Wih"].T, gs["Whh"].T, ub, k_head)
        P[name] = L
    return P


def kernel(x, edge_feat, edge_index, graph_ids, params):
    src_pad = jnp.pad(edge_index[0], (0, E_PAD - E)).reshape(NCHUNKS, CH)
    dst_pad = jnp.pad(edge_index[1], (0, E_PAD - E)).reshape(NCHUNKS, CH)
    ef_pad = jnp.pad(edge_feat, ((0, E_PAD - E), (0, 4)))
    x_pad = jnp.pad(x, ((0, 0), (0, 2)))
    oh = (graph_ids[:, None] == jnp.arange(16)[None, :]).astype(jnp.float32)

    P = _prep_weights(params)
    v, ssum, cnt = _node_init(x_pad, oh, *P["init"])
    kes = _ke_all(ef_pad, *P["ke"])

    sraw, craw = ssum, cnt
    for li, name in enumerate(("conv1", "conv2", "conv3")):
        L = P[name]
        s, s2s, dsup, s2mg = _sprep(sraw, craw, *L["sprep"])
        wa_stack, cvec, wd_stack, hbias, k_head = L["helper"]
        num, den = _helper_pass(v, oh, wa_stack, cvec, dsup, wd_stack,
                                hbias, k_head)
        parts = _edge_aggregate(v, kes[li], src_pad, dst_pad)
        vv = _gates(v, parts, oh, s2mg, *L["gates"])
        ss = _supdate(s, s2s, num, den, *L["supdate"])
        v, sraw, craw = vv, ss, jnp.ones((16, 128), jnp.float32)
    return v
